# baseline (speedup 1.0000x reference)
import jax
import jax.numpy as jnp
from jax.experimental import pallas as pl
from jax.experimental.pallas import tpu as pltpu

D_MODEL = 1024
SEQ_LEN = 2048
MAX_PROP = 256
AXES = 72
NPARAMS = 12
GLANE = 128
PROJ_PAD = NPARAMS * GLANE
SB = 256
CS = 32
UNROLL = 8
NBLK = SEQ_LEN // SB
ACC_ROWS = SEQ_LEN + MAX_PROP


def _softplus(x):
    return jnp.maximum(x, 0.0) + jnp.log1p(jnp.exp(-jnp.abs(x)))


def _body(hid_ref, wln_ref, bln_ref, wt_ref, b_ref,
          kin_ref, ab_ref, as_ref, gb_ref, gs_ref,
          acc_ref, st_ref, cf_ref):
    i = pl.program_id(0)

    @pl.when(i == 0)
    def _zero():
        acc_ref[:] = jnp.zeros_like(acc_ref)

    x = hid_ref[0]
    mean = jnp.mean(x, axis=1, keepdims=True)
    xc = x - mean
    var = jnp.mean(xc * xc, axis=1, keepdims=True)
    normed = xc * jax.lax.rsqrt(var + 1e-5) * wln_ref[:] + bln_ref[:]
    params = jnp.dot(normed, wt_ref[:],
                     preferred_element_type=jnp.float32) + b_ref[:]

    def g(j):
        return params[:, GLANE * j:GLANE * j + AXES]

    om_l = jnp.sqrt(_softplus(g(0)))
    decay_l = jnp.exp(-0.5 * _softplus(g(1)))
    om_a = jnp.sqrt(_softplus(g(2)))
    decay_a = jnp.exp(-0.5 * _softplus(g(3)))
    c = g(4)
    c_t = g(5)
    phi = g(6)
    phi_t = g(7)

    a_l = 2.0 * decay_l * jnp.cos(om_l)
    b_l = -(decay_l * decay_l)
    a_a = 2.0 * decay_a * jnp.cos(om_a)
    b_a = -(decay_a * decay_a)
    sin_om_l = jnp.sin(om_l)
    cos_om_l = jnp.cos(om_l)
    sin_om_a = jnp.sin(om_a)
    cos_om_a = jnp.cos(om_a)
    sin_phi = jnp.sin(phi)
    cos_phi = jnp.cos(phi)
    sin_phi_t = jnp.sin(phi_t)
    cos_phi_t = jnp.cos(phi_t)
    xl0 = c * sin_phi
    xl1 = decay_l * c * (sin_om_l * cos_phi + cos_om_l * sin_phi)
    xa0 = c_t * sin_phi_t
    xa1 = decay_a * c_t * (sin_om_a * cos_phi_t + cos_om_a * sin_phi_t)

    ab_ref[:] = g(8).T
    as_ref[:] = _softplus(g(9)).T
    gb_ref[:] = g(10).T
    gs_ref[:] = _softplus(g(11)).T

    st_ref[0] = xl0
    st_ref[1] = xl1
    st_ref[2] = xa0
    st_ref[3] = xa1
    cf_ref[0] = a_l
    cf_ref[1] = b_l
    cf_ref[2] = a_a
    cf_ref[3] = b_a

    base0 = i * SB
    n_groups = MAX_PROP // UNROLL

    def chunk(ci, carry):
        s0 = ci * CS
        al = cf_ref[0, pl.ds(s0, CS), :]
        bl = cf_ref[1, pl.ds(s0, CS), :]
        aa = cf_ref[2, pl.ds(s0, CS), :]
        ba = cf_ref[3, pl.ds(s0, CS), :]
        gbase = base0 + s0

        def group(tg, st):
            xlp, xlc, xap, xac = st
            start = gbase + tg * UNROLL
            w = acc_ref[pl.ds(start, CS + UNROLL), :]
            for j in range(UNROLL):
                val = xlp + xap
                w = w + jnp.pad(val, ((j, UNROLL - j), (0, 0)))
                xlp, xlc = xlc, al * xlc + bl * xlp
                xap, xac = xac, aa * xac + ba * xap
            acc_ref[pl.ds(start, CS + UNROLL), :] = w
            return (xlp, xlc, xap, xac)

        jax.lax.fori_loop(0, n_groups, group,
                          (st_ref[0, pl.ds(s0, CS), :],
                           st_ref[1, pl.ds(s0, CS), :],
                           st_ref[2, pl.ds(s0, CS), :],
                           st_ref[3, pl.ds(s0, CS), :]))
        return carry

    jax.lax.fori_loop(0, SB // CS, chunk, 0)

    @pl.when(i == NBLK - 1)
    def _emit_kin():
        def tile(c, carry):
            kin_ref[:, pl.ds(c * 128, 128)] = acc_ref[pl.ds(c * 128, 128), :].T
            return carry
        jax.lax.fori_loop(0, SEQ_LEN // 128, tile, 0)


def kernel(hidden_states, ln_weight, ln_bias, W, b):
    w_g = jnp.pad(W.reshape(NPARAMS, AXES, D_MODEL),
                  ((0, 0), (0, GLANE - AXES), (0, 0)))
    wt = w_g.reshape(PROJ_PAD, D_MODEL).T
    b_pad = jnp.pad(b.reshape(NPARAMS, AXES),
                    ((0, 0), (0, GLANE - AXES))).reshape(1, PROJ_PAD)
    wln = ln_weight.reshape(1, D_MODEL)
    bln = ln_bias.reshape(1, D_MODEL)

    outs = pl.pallas_call(
        _body,
        grid=(NBLK,),
        in_specs=[
            pl.BlockSpec((1, SB, D_MODEL), lambda i: (0, i, 0)),
            pl.BlockSpec((1, D_MODEL), lambda i: (0, 0)),
            pl.BlockSpec((1, D_MODEL), lambda i: (0, 0)),
            pl.BlockSpec((D_MODEL, PROJ_PAD), lambda i: (0, 0)),
            pl.BlockSpec((1, PROJ_PAD), lambda i: (0, 0)),
        ],
        out_specs=[
            pl.BlockSpec((AXES, SEQ_LEN), lambda i: (0, 0)),
            pl.BlockSpec((AXES, SB), lambda i: (0, i)),
            pl.BlockSpec((AXES, SB), lambda i: (0, i)),
            pl.BlockSpec((AXES, SB), lambda i: (0, i)),
            pl.BlockSpec((AXES, SB), lambda i: (0, i)),
        ],
        out_shape=[
            jax.ShapeDtypeStruct((AXES, SEQ_LEN), jnp.float32),
            jax.ShapeDtypeStruct((AXES, SEQ_LEN), jnp.float32),
            jax.ShapeDtypeStruct((AXES, SEQ_LEN), jnp.float32),
            jax.ShapeDtypeStruct((AXES, SEQ_LEN), jnp.float32),
            jax.ShapeDtypeStruct((AXES, SEQ_LEN), jnp.float32),
        ],
        scratch_shapes=[
            pltpu.VMEM((ACC_ROWS, AXES), jnp.float32),
            pltpu.VMEM((4, SB, AXES), jnp.float32),
            pltpu.VMEM((4, SB, AXES), jnp.float32),
        ],
        compiler_params=pltpu.CompilerParams(
            dimension_semantics=("arbitrary",),
        ),
    )(hidden_states, wln, bln, wt, b_pad)

    return tuple(outs)


# speedup vs baseline: 1.1429x; 1.1429x over previous
import jax
import jax.numpy as jnp
from jax.experimental import pallas as pl
from jax.experimental.pallas import tpu as pltpu

D_MODEL = 1024
SEQ_LEN = 2048
MAX_PROP = 256
AXES = 72
NPARAMS = 12
GLANE = 72
PROJ_PAD = 896
SB = 256
CS = 32
UNROLL = 8
NBLK = SEQ_LEN // SB
ACC_ROWS = SEQ_LEN + MAX_PROP


def _softplus(x):
    return jnp.maximum(x, 0.0) + jnp.log1p(jnp.exp(-jnp.abs(x)))


def _body(hid_ref, wln_ref, bln_ref, wt_ref, b_ref,
          kin_ref, ab_ref, as_ref, gb_ref, gs_ref,
          acc_ref, st_ref, cf_ref):
    i = pl.program_id(0)

    @pl.when(i == 0)
    def _zero():
        acc_ref[:] = jnp.zeros_like(acc_ref)

    x = hid_ref[0]
    mean = jnp.mean(x, axis=1, keepdims=True)
    xc = x - mean
    var = jnp.mean(xc * xc, axis=1, keepdims=True)
    normed = xc * jax.lax.rsqrt(var + 1e-5) * wln_ref[:] + bln_ref[:]
    params = jnp.dot(normed, wt_ref[:],
                     preferred_element_type=jnp.float32) + b_ref[:]

    def g(j):
        return params[:, GLANE * j:GLANE * j + AXES]

    om_l = jnp.sqrt(_softplus(g(0)))
    decay_l = jnp.exp(-0.5 * _softplus(g(1)))
    om_a = jnp.sqrt(_softplus(g(2)))
    decay_a = jnp.exp(-0.5 * _softplus(g(3)))
    c = g(4)
    c_t = g(5)
    phi = g(6)
    phi_t = g(7)

    a_l = 2.0 * decay_l * jnp.cos(om_l)
    b_l = -(decay_l * decay_l)
    a_a = 2.0 * decay_a * jnp.cos(om_a)
    b_a = -(decay_a * decay_a)
    sin_om_l = jnp.sin(om_l)
    cos_om_l = jnp.cos(om_l)
    sin_om_a = jnp.sin(om_a)
    cos_om_a = jnp.cos(om_a)
    sin_phi = jnp.sin(phi)
    cos_phi = jnp.cos(phi)
    sin_phi_t = jnp.sin(phi_t)
    cos_phi_t = jnp.cos(phi_t)
    xl0 = c * sin_phi
    xl1 = decay_l * c * (sin_om_l * cos_phi + cos_om_l * sin_phi)
    xa0 = c_t * sin_phi_t
    xa1 = decay_a * c_t * (sin_om_a * cos_phi_t + cos_om_a * sin_phi_t)

    ab_ref[:] = g(8).T
    as_ref[:] = _softplus(g(9)).T
    gb_ref[:] = g(10).T
    gs_ref[:] = _softplus(g(11)).T

    st_ref[0] = xl0
    st_ref[1] = xl1
    st_ref[2] = xa0
    st_ref[3] = xa1
    cf_ref[0] = a_l
    cf_ref[1] = b_l
    cf_ref[2] = a_a
    cf_ref[3] = b_a

    base0 = i * SB
    n_groups = MAX_PROP // UNROLL

    def chunk(ci, carry):
        s0 = ci * CS
        al = cf_ref[0, pl.ds(s0, CS), :]
        bl = cf_ref[1, pl.ds(s0, CS), :]
        aa = cf_ref[2, pl.ds(s0, CS), :]
        ba = cf_ref[3, pl.ds(s0, CS), :]
        gbase = base0 + s0

        def group(tg, st):
            xlp, xlc, xap, xac = st
            start = gbase + tg * UNROLL
            w = acc_ref[pl.ds(start, CS + UNROLL), :]
            for j in range(UNROLL):
                val = xlp + xap
                w = w + jnp.pad(val, ((j, UNROLL - j), (0, 0)))
                xlp, xlc = xlc, al * xlc + bl * xlp
                xap, xac = xac, aa * xac + ba * xap
            acc_ref[pl.ds(start, CS + UNROLL), :] = w
            return (xlp, xlc, xap, xac)

        jax.lax.fori_loop(0, n_groups, group,
                          (st_ref[0, pl.ds(s0, CS), :],
                           st_ref[1, pl.ds(s0, CS), :],
                           st_ref[2, pl.ds(s0, CS), :],
                           st_ref[3, pl.ds(s0, CS), :]))
        return carry

    jax.lax.fori_loop(0, SB // CS, chunk, 0)

    @pl.when(i == NBLK - 1)
    def _emit_kin():
        def tile(c, carry):
            kin_ref[:, pl.ds(c * 128, 128)] = acc_ref[pl.ds(c * 128, 128), :].T
            return carry
        jax.lax.fori_loop(0, SEQ_LEN // 128, tile, 0)


def kernel(hidden_states, ln_weight, ln_bias, W, b):
    wt = jnp.pad(W, ((0, PROJ_PAD - NPARAMS * AXES), (0, 0))).T
    b_pad = jnp.pad(b, (0, PROJ_PAD - NPARAMS * AXES)).reshape(1, PROJ_PAD)
    wln = ln_weight.reshape(1, D_MODEL)
    bln = ln_bias.reshape(1, D_MODEL)

    outs = pl.pallas_call(
        _body,
        grid=(NBLK,),
        in_specs=[
            pl.BlockSpec((1, SB, D_MODEL), lambda i: (0, i, 0)),
            pl.BlockSpec((1, D_MODEL), lambda i: (0, 0)),
            pl.BlockSpec((1, D_MODEL), lambda i: (0, 0)),
            pl.BlockSpec((D_MODEL, PROJ_PAD), lambda i: (0, 0)),
            pl.BlockSpec((1, PROJ_PAD), lambda i: (0, 0)),
        ],
        out_specs=[
            pl.BlockSpec((AXES, SEQ_LEN), lambda i: (0, 0)),
            pl.BlockSpec((AXES, SB), lambda i: (0, i)),
            pl.BlockSpec((AXES, SB), lambda i: (0, i)),
            pl.BlockSpec((AXES, SB), lambda i: (0, i)),
            pl.BlockSpec((AXES, SB), lambda i: (0, i)),
        ],
        out_shape=[
            jax.ShapeDtypeStruct((AXES, SEQ_LEN), jnp.float32),
            jax.ShapeDtypeStruct((AXES, SEQ_LEN), jnp.float32),
            jax.ShapeDtypeStruct((AXES, SEQ_LEN), jnp.float32),
            jax.ShapeDtypeStruct((AXES, SEQ_LEN), jnp.float32),
            jax.ShapeDtypeStruct((AXES, SEQ_LEN), jnp.float32),
        ],
        scratch_shapes=[
            pltpu.VMEM((ACC_ROWS, AXES), jnp.float32),
            pltpu.VMEM((4, SB, AXES), jnp.float32),
            pltpu.VMEM((4, SB, AXES), jnp.float32),
        ],
        compiler_params=pltpu.CompilerParams(
            dimension_semantics=("arbitrary",),
        ),
    )(hidden_states, wln, bln, wt, b_pad)

    return tuple(outs)
